# baseline (speedup 1.0000x reference)
import functools

import jax
import jax.numpy as jnp
from jax import lax
from jax.experimental import pallas as pl
from jax.experimental.pallas import tpu as pltpu
from jax.experimental.pallas import tpu_sc as plsc

V = 1000000
D = 32
R = 16384
S = 50
NC, NS = 2, 16
NW = NC * NS
RPW = R // NW
TCW = RPW // 128
SCH = 2
NU = TCW * (S // SCH)
NPAIR = NU // 2
UROWS = SCH * 128

_mesh = plsc.VectorSubcoreMesh(core_axis_name="c", subcore_axis_name="s")


@functools.partial(
    pl.kernel,
    mesh=_mesh,
    out_type=jax.ShapeDtypeStruct((S * D // 8, R // 128, 8, 128), jnp.float32),
    scratch_types=[
        pltpu.VMEM((RPW, S), jnp.int32),
        pltpu.VMEM((UROWS,), jnp.int32),
        pltpu.VMEM((UROWS,), jnp.int32),
        pltpu.VMEM((UROWS, D), jnp.float32),
        pltpu.VMEM((UROWS, D), jnp.float32),
        pltpu.VMEM((SCH * 4, 8, 128), jnp.float32),
        pltpu.VMEM((SCH * 4, 8, 128), jnp.float32),
        pltpu.SemaphoreType.DMA,
        pltpu.SemaphoreType.DMA,
        pltpu.SemaphoreType.DMA,
        pltpu.SemaphoreType.DMA,
    ],
    compiler_params=pltpu.CompilerParams(
        use_tc_tiling_on_sc=False, needs_layout_passes=False
    ),
)
def _emb_lookup(table_hbm, idx_hbm, out_hbm, idx_s, blk_a, blk_b,
                gbuf_a, gbuf_b, stage_a, stage_b,
                gsem_a, gsem_b, ssem_a, ssem_b):
    wid = lax.axis_index("s") * NC + lax.axis_index("c")
    wbase = wid * RPW
    pltpu.sync_copy(idx_hbm.at[pl.ds(wbase, RPW)], idx_s)

    blks = (blk_a, blk_b)
    gbufs = (gbuf_a, gbuf_b)
    stages = (stage_a, stage_b)
    gsems = (gsem_a, gsem_b)
    ssems = (ssem_a, ssem_b)

    lane = lax.iota(jnp.int32, 16)
    def unit_coords(u):
        tc_l = u // (NU // TCW)
        s0 = (u % (NU // TCW)) * SCH
        return tc_l, s0

    def prep_and_gather(u, bank):
        tc_l, s0 = unit_coords(u)
        for sc in range(SCH):
            s_col = jnp.full((16,), s0 + sc, jnp.int32)
            for lg in range(8):
                rows = tc_l * 128 + lg * 16 + lane
                iv = plsc.load_gather(idx_s, [rows, s_col])
                blks[bank][pl.ds(sc * 128 + lg * 16, 16)] = iv
        for h in range(UROWS // 128):
            pltpu.async_copy(
                table_hbm.at[blks[bank].at[pl.ds(h * 128, 128)]],
                gbufs[bank].at[pl.ds(h * 128, 128)],
                gsems[bank],
            )

    def wait_gathers(u, bank):
        pltpu.make_async_copy(
            table_hbm.at[pl.ds(0, UROWS)], gbufs[bank], gsems[bank]
        ).wait()

    def extract(u, bank):
        gbuf = gbufs[bank]
        stage = stages[bank]
        for sc in range(SCH):
            for lg in range(8):
                rows = jnp.asarray(sc * 128 + lg * 16, jnp.int32) + lane
                for d in range(D):
                    col = jnp.full((16,), d, jnp.int32)
                    v = plsc.load_gather(gbuf, [rows, col])
                    stage[sc * 4 + d // 8, d % 8, pl.ds(lg * 16, 16)] = v

    def start_store(u, bank):
        tc_l, s0 = unit_coords(u)
        pltpu.async_copy(
            stages[bank],
            out_hbm.at[pl.ds(4 * s0, SCH * 4), wid * TCW + tc_l],
            ssems[bank],
        )

    def wait_store(u, bank):
        tc_l, s0 = unit_coords(u)
        pltpu.make_async_copy(
            stages[bank],
            out_hbm.at[pl.ds(4 * s0, SCH * 4), wid * TCW + tc_l],
            ssems[bank],
        ).wait()

    prep_and_gather(0, 0)

    def body(t, carry):
        u_a = 2 * t
        u_b = u_a + 1

        prep_and_gather(u_b, 1)
        wait_gathers(u_a, 0)

        @pl.when(t > 0)
        def _():
            wait_store(u_a - 2, 0)

        extract(u_a, 0)
        start_store(u_a, 0)

        @pl.when(t < NPAIR - 1)
        def _():
            prep_and_gather(u_a + 2, 0)

        wait_gathers(u_b, 1)

        @pl.when(t > 0)
        def _():
            wait_store(u_b - 2, 1)

        extract(u_b, 1)
        start_store(u_b, 1)
        return carry

    lax.fori_loop(0, NPAIR, body, 0)
    wait_store(NU - 2, 0)
    wait_store(NU - 1, 1)


def kernel(input_, weight):
    out4 = _emb_lookup(weight, input_.astype(jnp.int32))
    o = out4.reshape(S, 4, R // 128, 8, 128)
    o = o.transpose(2, 4, 0, 1, 3)
    return o.reshape(R, S, D)
